# Initial kernel scaffold; baseline (speedup 1.0000x reference)
import jax
import jax.numpy as jnp
from jax.experimental import pallas as pl


def kernel(inputs, edge_index, W, b):
    raise NotImplementedError("write your pallas kernel here")


# speedup vs baseline: 3.6820x; 3.6820x over previous
import functools

import jax
import jax.numpy as jnp
from jax import lax
from jax.experimental import pallas as pl
from jax.experimental.pallas import tpu as pltpu
from jax.experimental.pallas import tpu_sc as plsc

N_NODES = 10000
D_FEAT = 128
N_EDGES = 320000

NC = 2
NS = 16
NW = NC * NS
CHUNK = 128

ROWS_PER_TILE = 624
REM_ROW0 = NS * ROWS_PER_TILE
REM_ROWS = N_NODES - REM_ROW0
CHUNKS_PER_TILE = -(-N_EDGES // (NW * CHUNK))
EDGES_PER_TILE = CHUNKS_PER_TILE * CHUNK
E_PAD = NW * EDGES_PER_TILE
ACC_ROWS = N_NODES + 8


def _sc_body(x_hbm, src_hbm, dst_hbm, zeros_hbm, out_hbm,
             acc, src_v, dst_v, rows_v, sem):
    cid = lax.axis_index("c")
    sid = lax.axis_index("s")
    row0 = sid * ROWS_PER_TILE

    @pl.when(cid == 0)
    def _():
        pltpu.sync_copy(x_hbm.at[pl.ds(row0, ROWS_PER_TILE)],
                        acc.at[pl.ds(row0, ROWS_PER_TILE)])

    @pl.when(cid == 1)
    def _():
        pltpu.sync_copy(zeros_hbm, acc.at[pl.ds(row0, ROWS_PER_TILE)])

    @pl.when((cid == 0) & (sid == NS - 1))
    def _():
        pltpu.sync_copy(x_hbm.at[pl.ds(REM_ROW0, REM_ROWS)],
                        acc.at[pl.ds(REM_ROW0, REM_ROWS)])

    @pl.when((cid == 1) & (sid == NS - 1))
    def _():
        pltpu.sync_copy(zeros_hbm.at[pl.ds(0, REM_ROWS)],
                        acc.at[pl.ds(REM_ROW0, REM_ROWS)])

    plsc.subcore_barrier()

    wid = sid * NC + cid
    base = wid * EDGES_PER_TILE

    def step(i, carry):
        off = base + i * CHUNK
        pltpu.sync_copy(src_hbm.at[pl.ds(off, CHUNK)], src_v)
        pltpu.sync_copy(dst_hbm.at[pl.ds(off, CHUNK)], dst_v)
        pltpu.async_copy(x_hbm.at[src_v], rows_v, sem).wait()
        pltpu.sync_copy(rows_v, acc.at[dst_v], add=True)
        return carry

    lax.fori_loop(0, CHUNKS_PER_TILE, step, 0)

    plsc.subcore_barrier()

    pltpu.sync_copy(acc.at[pl.ds(row0, ROWS_PER_TILE)],
                    out_hbm.at[pl.ds(cid * N_NODES + row0, ROWS_PER_TILE)])

    @pl.when(sid == NS - 1)
    def _():
        pltpu.sync_copy(acc.at[pl.ds(REM_ROW0, REM_ROWS)],
                        out_hbm.at[pl.ds(cid * N_NODES + REM_ROW0, REM_ROWS)])


@jax.jit
def _sc_aggregate(x, src, dst, zeros):
    mesh = plsc.VectorSubcoreMesh(core_axis_name="c", subcore_axis_name="s")
    k = pl.kernel(
        _sc_body,
        out_type=jax.ShapeDtypeStruct((NC * N_NODES, D_FEAT), jnp.float32),
        mesh=mesh,
        scratch_types=[
            pltpu.VMEM_SHARED((ACC_ROWS, D_FEAT), jnp.float32),
            pltpu.VMEM((CHUNK,), jnp.int32),
            pltpu.VMEM((CHUNK,), jnp.int32),
            pltpu.VMEM((CHUNK, D_FEAT), jnp.float32),
            pltpu.SemaphoreType.DMA,
        ],
    )
    return k(x, src, dst, zeros)


def _mm_body(p0_ref, p1_ref, w_ref, b_ref, out_ref):
    h = p0_ref[...] + p1_ref[...]
    out = jnp.dot(h, w_ref[...], preferred_element_type=jnp.float32)
    out_ref[...] = jnp.maximum(out + b_ref[...], 0.0)


BLOCK_M = 1000


@jax.jit
def _mm(p0, p1, W, b2d):
    grid = (N_NODES // BLOCK_M,)
    return pl.pallas_call(
        _mm_body,
        grid=grid,
        in_specs=[
            pl.BlockSpec((BLOCK_M, D_FEAT), lambda i: (i, 0)),
            pl.BlockSpec((BLOCK_M, D_FEAT), lambda i: (i, 0)),
            pl.BlockSpec((D_FEAT, D_FEAT), lambda i: (0, 0)),
            pl.BlockSpec((1, D_FEAT), lambda i: (0, 0)),
        ],
        out_specs=pl.BlockSpec((BLOCK_M, D_FEAT), lambda i: (i, 0)),
        out_shape=jax.ShapeDtypeStruct((N_NODES, D_FEAT), jnp.float32),
    )(p0, p1, W, b2d)


def kernel(inputs, edge_index, W, b):
    src = edge_index[0].astype(jnp.int32)
    dst = edge_index[1].astype(jnp.int32)
    pad = E_PAD - N_EDGES
    src = jnp.concatenate([src, jnp.zeros((pad,), jnp.int32)])
    dst = jnp.concatenate([dst, jnp.full((pad,), N_NODES, jnp.int32)])
    zeros = jnp.zeros((ROWS_PER_TILE, D_FEAT), jnp.float32)
    parts = _sc_aggregate(inputs, src, dst, zeros)
    return _mm(parts[:N_NODES], parts[N_NODES:], W, b.reshape(1, D_FEAT))
